# Initial kernel scaffold; baseline (speedup 1.0000x reference)
import jax
import jax.numpy as jnp
from jax.experimental import pallas as pl


def kernel(directions, perturbations, W_in0, W_in1, W_hid0_log, w_out_log, length_scale):
    raise NotImplementedError("write your pallas kernel here")


# speedup vs baseline: 2.6285x; 2.6285x over previous
import functools

import jax
import jax.numpy as jnp
from jax.experimental import pallas as pl
from jax.experimental.pallas import tpu as pltpu

_C = 5
_K = 4
_W = 256
_BT = 512

_bf16 = jnp.bfloat16
_f32 = jnp.float32


def _body(ls_ref, d_ref, u_ref, w0_ref, w1_ref, w0t_ref, w1t_ref,
          exph_ref, expht_ref, w_ref, out_ref):
    w = w_ref[...] * ls_ref[0]

    d = d_ref[...]
    dx = d[:, 0:1]
    dy = d[:, 1:2]
    dz = d[:, 2:3]

    u_all = u_ref[...]
    W0 = w0_ref[...]
    W1 = w1_ref[...]
    expH = exph_ref[...]
    expHT = expht_ref[...]
    W0T = w0t_ref[...]
    W1T = w1t_ref[...]

    dots = []
    vx_all, vy_all, vz_all = [], [], []
    for c in range(_C):
        ub = u_all[:, 3 * c:3 * c + 3].astype(_bf16)

        Z1 = jnp.dot(ub, W0, preferred_element_type=_f32)
        H1b = jnp.maximum(Z1, 0.0).astype(_bf16)
        Z2 = (jnp.dot(H1b, expH, preferred_element_type=_f32)
              + jnp.dot(ub, W1, preferred_element_type=_f32))

        Gb = jnp.where(Z2 > 0.0, w, 0.0).astype(_bf16)
        T = jnp.dot(Gb, expHT, preferred_element_type=_f32)
        Tb = jnp.where(Z1 > 0.0, T, 0.0).astype(_bf16)
        vert = (jnp.dot(Gb, W1T, preferred_element_type=_f32)
                + jnp.dot(Tb, W0T, preferred_element_type=_f32))
        vx = vert[:, 0:1]
        vy = vert[:, 1:2]
        vz = vert[:, 2:3]

        dots.append((dx * vx + dy * vy) + dz * vz)
        vx_all.append(vx)
        vy_all.append(vy)
        vz_all.append(vz)

    ranks = []
    for c in range(_C):
        r = jnp.zeros((d.shape[0], 1), jnp.int32)
        for c2 in range(_C):
            if c2 == c:
                continue
            beats = dots[c2] > dots[c]
            if c2 < c:
                beats = beats | (dots[c2] == dots[c])
            r = r + beats.astype(jnp.int32)
        ranks.append(r)

    cols = []
    for q in range(_K):
        ox = jnp.zeros_like(vx_all[0])
        oy = jnp.zeros_like(ox)
        oz = jnp.zeros_like(ox)
        for c in range(_C):
            sel = ranks[c] == q
            ox = ox + jnp.where(sel, vx_all[c], 0.0)
            oy = oy + jnp.where(sel, vy_all[c], 0.0)
            oz = oz + jnp.where(sel, vz_all[c], 0.0)
        cols.extend([ox, oy, oz])
    out_ref[...] = jnp.concatenate(cols, axis=1)


@functools.partial(jax.jit, static_argnames=())
def kernel(directions, perturbations, W_in0, W_in1, W_hid0_log, w_out_log,
           length_scale):
    B = directions.shape[0]
    pert = jnp.concatenate(
        [jnp.zeros((1, 3), directions.dtype), perturbations], axis=0)
    cand = directions[:, None, :] + pert[None, :, :]
    u = cand / jnp.sqrt(jnp.sum(cand * cand, axis=-1, keepdims=True))
    u_flat = u.reshape(B, _C * 3)

    ls = jnp.reshape(length_scale, (1,)).astype(_f32)
    expH = jnp.exp(W_hid0_log)
    expH_b = expH.astype(_bf16)
    expHT_b = expH.T.astype(_bf16)
    W0_b = W_in0.astype(_bf16)
    W1_b = W_in1.astype(_bf16)
    W0T_b = W_in0.T.astype(_bf16)
    W1T_b = W_in1.T.astype(_bf16)
    w_out = jnp.reshape(jnp.exp(w_out_log), (1, _W))

    grid = (B // _BT,)
    out = pl.pallas_call(
        _body,
        grid=grid,
        in_specs=[
            pl.BlockSpec(memory_space=pltpu.SMEM),
            pl.BlockSpec((_BT, 3), lambda i: (i, 0)),
            pl.BlockSpec((_BT, _C * 3), lambda i: (i, 0)),
            pl.BlockSpec((3, _W), lambda i: (0, 0)),
            pl.BlockSpec((3, _W), lambda i: (0, 0)),
            pl.BlockSpec((_W, 3), lambda i: (0, 0)),
            pl.BlockSpec((_W, 3), lambda i: (0, 0)),
            pl.BlockSpec((_W, _W), lambda i: (0, 0)),
            pl.BlockSpec((_W, _W), lambda i: (0, 0)),
            pl.BlockSpec((1, _W), lambda i: (0, 0)),
        ],
        out_specs=pl.BlockSpec((_BT, 3 * _K), lambda i: (i, 0)),
        out_shape=jax.ShapeDtypeStruct((B, 3 * _K), jnp.float32),
    )(ls, directions, u_flat, W0_b, W1_b, W0T_b, W1T_b, expH_b, expHT_b,
      w_out)
    return out.reshape(B, _K, 3)
